# baseline (speedup 1.0000x reference)
import functools

import jax
import jax.numpy as jnp
from jax.experimental import pallas as pl
from jax.experimental.pallas import tpu as pltpu

_K_IN = 4
_K_OUT = 4


def _round_up(x, m):
    return ((x + m - 1) // m) * m


def _pipe_kernel(x_hbm, w_ref, o_hbm, x_buf, o_buf, in_sem, out_sem,
                 *, block, n_steps):
    c = pl.program_id(0)
    base = c * (n_steps * block)

    def dma_in(slot, step):
        pltpu.make_async_copy(
            x_hbm.at[pl.ds(base + step * block, block), :],
            x_buf.at[slot], in_sem.at[slot]).start()

    def wait_in(slot):
        pltpu.make_async_copy(
            x_hbm.at[pl.ds(0, block), :],
            x_buf.at[slot], in_sem.at[slot]).wait()

    def dma_out(slot, step):
        pltpu.make_async_copy(
            o_buf.at[slot],
            o_hbm.at[pl.ds(base + step * block, block), :],
            out_sem.at[slot]).start()

    def wait_out(slot):
        pltpu.make_async_copy(
            o_buf.at[slot],
            o_hbm.at[pl.ds(0, block), :],
            out_sem.at[slot]).wait()

    for s in range(min(_K_IN, n_steps)):
        dma_in(s, s)

    def body(step, carry):
        cur_in = jax.lax.rem(step, _K_IN)
        cur_out = jax.lax.rem(step, _K_OUT)
        wait_in(cur_in)

        @pl.when(step >= _K_OUT)
        def _():
            wait_out(cur_out)

        o_buf[cur_out] = jnp.dot(
            x_buf[cur_in].astype(jnp.bfloat16), w_ref[...],
            preferred_element_type=jnp.float32)
        dma_out(cur_out, step)

        @pl.when(step + _K_IN < n_steps)
        def _():
            dma_in(cur_in, step + _K_IN)

        return carry

    jax.lax.fori_loop(0, n_steps, body, 0)
    for d in range(min(_K_OUT, n_steps), 0, -1):
        wait_out(jax.lax.rem(n_steps - d, _K_OUT))


def kernel(X, W):
    N, D = X.shape
    D2, H = W.shape
    assert D == D2
    out_dtype = X.dtype

    Wb = W.astype(jnp.bfloat16)

    block = 1024
    n_pad = _round_up(N, 2 * block)
    Xp = X if n_pad == N else jnp.pad(X, ((0, n_pad - N), (0, 0)))
    n_steps = n_pad // (2 * block)

    kern = functools.partial(_pipe_kernel, block=block, n_steps=n_steps)
    out = pl.pallas_call(
        kern,
        out_shape=jax.ShapeDtypeStruct((n_pad, H), out_dtype),
        grid=(2,),
        in_specs=[
            pl.BlockSpec(memory_space=pl.ANY),
            pl.BlockSpec((D, H), lambda c: (0, 0)),
        ],
        out_specs=pl.BlockSpec(memory_space=pl.ANY),
        scratch_shapes=[
            pltpu.VMEM((_K_IN, block, D), jnp.float32),
            pltpu.VMEM((_K_OUT, block, H), jnp.float32),
            pltpu.SemaphoreType.DMA((_K_IN,)),
            pltpu.SemaphoreType.DMA((_K_OUT,)),
        ],
        compiler_params=pltpu.CompilerParams(
            dimension_semantics=("parallel",),
            vmem_limit_bytes=57 * 1024 * 1024,
        ),
    )(Xp, Wb)
    return out[:N] if n_pad != N else out


# speedup vs baseline: 1.0353x; 1.0353x over previous
import functools

import jax
import jax.numpy as jnp
from jax.experimental import pallas as pl
from jax.experimental.pallas import tpu as pltpu

_K_IN = 3
_K_OUT = 4


def _round_up(x, m):
    return ((x + m - 1) // m) * m


def _pipe_kernel(x_hbm, w_ref, o_hbm, x_buf, o_buf, in_sem, out_sem,
                 *, block, n_steps):
    c = pl.program_id(0)
    base = c * (n_steps * block)

    def dma_in(slot, step):
        pltpu.make_async_copy(
            x_hbm.at[pl.ds(base + step * block, block), :],
            x_buf.at[slot], in_sem.at[slot]).start()

    def wait_in(slot):
        pltpu.make_async_copy(
            x_hbm.at[pl.ds(0, block), :],
            x_buf.at[slot], in_sem.at[slot]).wait()

    def dma_out(slot, step):
        pltpu.make_async_copy(
            o_buf.at[slot],
            o_hbm.at[pl.ds(base + step * block, block), :],
            out_sem.at[slot]).start()

    def wait_out(slot):
        pltpu.make_async_copy(
            o_buf.at[slot],
            o_hbm.at[pl.ds(0, block), :],
            out_sem.at[slot]).wait()

    for s in range(min(_K_IN, n_steps)):
        dma_in(s, s)

    def body(step, carry):
        cur_in = jax.lax.rem(step, _K_IN)
        cur_out = jax.lax.rem(step, _K_OUT)
        wait_in(cur_in)

        @pl.when(step >= _K_OUT)
        def _():
            wait_out(cur_out)

        o_buf[cur_out] = jnp.dot(
            x_buf[cur_in].astype(jnp.bfloat16), w_ref[...],
            preferred_element_type=jnp.float32)
        dma_out(cur_out, step)

        @pl.when(step + _K_IN < n_steps)
        def _():
            dma_in(cur_in, step + _K_IN)

        return carry

    jax.lax.fori_loop(0, n_steps, body, 0)
    for d in range(min(_K_OUT, n_steps), 0, -1):
        wait_out(jax.lax.rem(n_steps - d, _K_OUT))


def kernel(X, W):
    N, D = X.shape
    D2, H = W.shape
    assert D == D2
    out_dtype = X.dtype

    Wb = W.astype(jnp.bfloat16)

    block = 2048
    n_pad = _round_up(N, 2 * block)
    Xp = X if n_pad == N else jnp.pad(X, ((0, n_pad - N), (0, 0)))
    n_steps = n_pad // (2 * block)

    kern = functools.partial(_pipe_kernel, block=block, n_steps=n_steps)
    out = pl.pallas_call(
        kern,
        out_shape=jax.ShapeDtypeStruct((n_pad, H), out_dtype),
        grid=(2,),
        in_specs=[
            pl.BlockSpec(memory_space=pl.ANY),
            pl.BlockSpec((D, H), lambda c: (0, 0)),
        ],
        out_specs=pl.BlockSpec(memory_space=pl.ANY),
        scratch_shapes=[
            pltpu.VMEM((_K_IN, block, D), jnp.float32),
            pltpu.VMEM((_K_OUT, block, H), jnp.float32),
            pltpu.SemaphoreType.DMA((_K_IN,)),
            pltpu.SemaphoreType.DMA((_K_OUT,)),
        ],
        compiler_params=pltpu.CompilerParams(
            dimension_semantics=("parallel",),
            vmem_limit_bytes=57 * 1024 * 1024,
        ),
    )(Xp, Wb)
    return out[:N] if n_pad != N else out
